# baseline (speedup 1.0000x reference)
import functools

import jax
import jax.numpy as jnp
from jax import lax
from jax.experimental import pallas as pl
from jax.experimental.pallas import tpu as pltpu
from jax.experimental.pallas import tpu_sc as plsc

D = 32
H = 16
B = 16384
L = 16
NC, NS = 2, 16
NW = NC * NS
BPW = B // NW
CHUNK = 128
NCHUNK = BPW // CHUNK
NBLK2 = BPW // (2 * L)


def _gmf_body(cell_idx_hbm, gene_idx_hbm, cell_tab, gene_tab,
              w1b_hbm, b1b_hbm, w2b_hbm, b2b_hbm, out_hbm,
              idx_c, idx_g, cell_v, gene_v,
              w1v, b1v, w2v, b2v, out_v, sem_c, sem_g):
    wid = lax.axis_index("s") * NC + lax.axis_index("c")
    base = wid * BPW

    pltpu.sync_copy(cell_idx_hbm.at[pl.ds(base, BPW)], idx_c)
    pltpu.sync_copy(gene_idx_hbm.at[pl.ds(base, BPW)], idx_g)
    pltpu.sync_copy(w1b_hbm, w1v)
    pltpu.sync_copy(b1b_hbm, b1v)
    pltpu.sync_copy(w2b_hbm, w2v)
    pltpu.sync_copy(b2b_hbm, b2v)

    def fire(d, carry):
        for c in range(NCHUNK):
            isl = pl.ds(c * CHUNK, CHUNK)
            dsl = pl.ds(c * CHUNK, CHUNK)
            pltpu.async_copy(cell_tab.at[d].at[idx_c.at[isl]],
                             cell_v.at[d, dsl], sem_c)
            pltpu.async_copy(gene_tab.at[d].at[idx_g.at[isl]],
                             gene_v.at[d, dsl], sem_g)
        @pl.when(d >= 1)
        def _():
            pltpu.make_async_copy(cell_tab.at[0].at[pl.ds(0, BPW)],
                                  cell_v.at[0, pl.ds(0, BPW)], sem_c).wait()
            pltpu.make_async_copy(gene_tab.at[0].at[pl.ds(0, BPW)],
                                  gene_v.at[0, pl.ds(0, BPW)], sem_g).wait()
        return carry

    lax.fori_loop(0, D, fire, 0)
    pltpu.make_async_copy(cell_tab.at[0].at[pl.ds(0, BPW)],
                          cell_v.at[0, pl.ds(0, BPW)], sem_c).wait()
    pltpu.make_async_copy(gene_tab.at[0].at[pl.ds(0, BPW)],
                          gene_v.at[0, pl.ds(0, BPW)], sem_g).wait()

    lanes = lax.iota(jnp.int32, L)
    del lanes

    def blk_body(j, carry):
        r0 = pl.ds(2 * j * L, L)
        r1 = pl.ds((2 * j + 1) * L, L)
        h0 = [b1v[pl.ds(k * L, L)] for k in range(H)]
        h1 = list(h0)
        for d in range(D):
            p0 = cell_v[d, r0] * gene_v[d, r0]
            p1 = cell_v[d, r1] * gene_v[d, r1]
            for k in range(H):
                w = w1v[pl.ds((d * H + k) * L, L)]
                h0[k] = h0[k] + p0 * w
                h1[k] = h1[k] + p1 * w
        acc0 = b2v[pl.ds(0, L)]
        acc1 = acc0
        for k in range(H):
            w = w2v[pl.ds(k * L, L)]
            acc0 = acc0 + jnp.maximum(h0[k], 0.0) * w
            acc1 = acc1 + jnp.maximum(h1[k], 0.0) * w
        out_v[r0] = 1.0 / (1.0 + jnp.exp(-acc0))
        out_v[r1] = 1.0 / (1.0 + jnp.exp(-acc1))
        return carry

    lax.fori_loop(0, NBLK2, blk_body, 0)

    pltpu.sync_copy(out_v, out_hbm.at[pl.ds(base, BPW)])


@functools.partial(
    pl.kernel,
    out_type=jax.ShapeDtypeStruct((B,), jnp.float32),
    mesh=plsc.VectorSubcoreMesh(core_axis_name="c", subcore_axis_name="s"),
    compiler_params=pltpu.CompilerParams(needs_layout_passes=False,
                                         use_tc_tiling_on_sc=False),
    scratch_types=[
        pltpu.VMEM((BPW,), jnp.int32),
        pltpu.VMEM((BPW,), jnp.int32),
        pltpu.VMEM((D, BPW), jnp.float32),
        pltpu.VMEM((D, BPW), jnp.float32),
        pltpu.VMEM((D * H * L,), jnp.float32),
        pltpu.VMEM((H * L,), jnp.float32),
        pltpu.VMEM((H * L,), jnp.float32),
        pltpu.VMEM((L,), jnp.float32),
        pltpu.VMEM((BPW,), jnp.float32),
        pltpu.SemaphoreType.DMA,
        pltpu.SemaphoreType.DMA,
    ],
)
def _gmf_kernel(*refs):
    _gmf_body(*refs)


def kernel(cell_indices, gene_indices, emb_cell, emb_gene, W1, b1, W2, b2):
    cellT = emb_cell.T
    geneT = emb_gene.T
    w1b = jnp.broadcast_to(W1.reshape(D, H, 1), (D, H, L)).reshape(-1)
    b1b = jnp.broadcast_to(b1.reshape(H, 1), (H, L)).reshape(-1)
    w2b = jnp.broadcast_to(W2.reshape(H, 1), (H, L)).reshape(-1)
    b2b = jnp.broadcast_to(b2.reshape(1, 1), (1, L)).reshape(-1)
    out = _gmf_kernel(cell_indices.astype(jnp.int32),
                      gene_indices.astype(jnp.int32),
                      cellT, geneT, w1b, b1b, w2b, b2b)
    return out.reshape(B, 1)


# speedup vs baseline: 4.6378x; 4.6378x over previous
import functools

import jax
import jax.numpy as jnp
from jax import lax
from jax.experimental import pallas as pl
from jax.experimental.pallas import tpu as pltpu
from jax.experimental.pallas import tpu_sc as plsc

D = 32
H = 16
B = 16384
L = 16
NC, NS = 2, 16
NW = NC * NS
BPW = B // NW
SB = 128
NSB = BPW // SB
NBLK2 = SB // (2 * L)


def _gmf_body(cell_idx_hbm, gene_idx_hbm, cell_tab, gene_tab,
              w1b_hbm, b1b_hbm, w2b_hbm, b2b_hbm, out_hbm,
              idx_c, idx_g,
              cell_b0, gene_b0, cell_b1, gene_b1,
              w1v, b1v, w2v, b2v, out_v, sem0, sem1):
    wid = lax.axis_index("s") * NC + lax.axis_index("c")
    base = wid * BPW

    pltpu.sync_copy(cell_idx_hbm.at[pl.ds(base, BPW)], idx_c)
    pltpu.sync_copy(gene_idx_hbm.at[pl.ds(base, BPW)], idx_g)
    pltpu.sync_copy(w1b_hbm, w1v)
    pltpu.sync_copy(b1b_hbm, b1v)
    pltpu.sync_copy(w2b_hbm, w2v)
    pltpu.sync_copy(b2b_hbm, b2v)

    lanes = lax.iota(jnp.int32, L)
    bufs = ((cell_b0, gene_b0, sem0), (cell_b1, gene_b1, sem1))

    def fire(sb):
        cb, gb, sem = bufs[sb % 2]
        src = pl.ds(sb * SB, SB)
        return (pltpu.async_copy(cell_tab.at[idx_c.at[src]], cb, sem),
                pltpu.async_copy(gene_tab.at[idx_g.at[src]], gb, sem))

    inflight = fire(0)
    for sb in range(NSB):
        cb, gb, _ = bufs[sb % 2]
        for cp in inflight:
            cp.wait()
        if sb + 1 < NSB:
            inflight = fire(sb + 1)

        def blk_body(j, carry, sb=sb, cb=cb, gb=gb):
            r0 = pl.ds(sb * SB + 2 * j * L, L)
            r1 = pl.ds(sb * SB + (2 * j + 1) * L, L)
            rows0 = lanes + 2 * j * L
            rows1 = rows0 + L
            h0 = [b1v[pl.ds(k * L, L)] for k in range(H)]
            h1 = list(h0)
            for d in range(D):
                dcol = jnp.full((L,), d, jnp.int32)
                p0 = (plsc.load_gather(cb, [rows0, dcol])
                      * plsc.load_gather(gb, [rows0, dcol]))
                p1 = (plsc.load_gather(cb, [rows1, dcol])
                      * plsc.load_gather(gb, [rows1, dcol]))
                for k in range(H):
                    w = w1v[pl.ds((d * H + k) * L, L)]
                    h0[k] = h0[k] + p0 * w
                    h1[k] = h1[k] + p1 * w
            acc0 = b2v[pl.ds(0, L)]
            acc1 = acc0
            for k in range(H):
                w = w2v[pl.ds(k * L, L)]
                acc0 = acc0 + jnp.maximum(h0[k], 0.0) * w
                acc1 = acc1 + jnp.maximum(h1[k], 0.0) * w
            out_v[r0] = 1.0 / (1.0 + jnp.exp(-acc0))
            out_v[r1] = 1.0 / (1.0 + jnp.exp(-acc1))
            return carry

        lax.fori_loop(0, NBLK2, blk_body, 0)

    pltpu.sync_copy(out_v, out_hbm.at[pl.ds(base, BPW)])


@functools.partial(
    pl.kernel,
    out_type=jax.ShapeDtypeStruct((B,), jnp.float32),
    mesh=plsc.VectorSubcoreMesh(core_axis_name="c", subcore_axis_name="s"),
    compiler_params=pltpu.CompilerParams(needs_layout_passes=False),
    scratch_types=[
        pltpu.VMEM((BPW,), jnp.int32),
        pltpu.VMEM((BPW,), jnp.int32),
        pltpu.VMEM((SB, 128), jnp.float32),
        pltpu.VMEM((SB, 128), jnp.float32),
        pltpu.VMEM((SB, 128), jnp.float32),
        pltpu.VMEM((SB, 128), jnp.float32),
        pltpu.VMEM((D * H * L,), jnp.float32),
        pltpu.VMEM((H * L,), jnp.float32),
        pltpu.VMEM((H * L,), jnp.float32),
        pltpu.VMEM((L,), jnp.float32),
        pltpu.VMEM((BPW,), jnp.float32),
        pltpu.SemaphoreType.DMA,
        pltpu.SemaphoreType.DMA,
    ],
)
def _gmf_kernel(*refs):
    _gmf_body(*refs)


def kernel(cell_indices, gene_indices, emb_cell, emb_gene, W1, b1, W2, b2):
    cellp = jnp.pad(emb_cell, ((0, 0), (0, 128 - D)))
    genep = jnp.pad(emb_gene, ((0, 0), (0, 128 - D)))
    w1b = jnp.broadcast_to(W1.reshape(D, H, 1), (D, H, L)).reshape(-1)
    b1b = jnp.broadcast_to(b1.reshape(H, 1), (H, L)).reshape(-1)
    w2b = jnp.broadcast_to(W2.reshape(H, 1), (H, L)).reshape(-1)
    b2b = jnp.broadcast_to(b2.reshape(1, 1), (1, L)).reshape(-1)
    out = _gmf_kernel(cell_indices.astype(jnp.int32),
                      gene_indices.astype(jnp.int32),
                      cellp, genep, w1b, b1b, w2b, b2b)
    return out.reshape(B, 1)
